# baseline (speedup 1.0000x reference)
import jax
import jax.numpy as jnp
from jax import lax
from jax.experimental import pallas as pl
from jax.experimental.pallas import tpu as pltpu
from jax.experimental.pallas import tpu_sc as plsc

N_CLS = 1000
DIM = 768
N_CTX = 4
SUF = 72
PRE_W = DIM
CTX_W = N_CTX * DIM
SUF_W = SUF * DIM
ROW_W = PRE_W + CTX_W + SUF_W

NW = 32
BLK = 32
TAIL = N_CLS - BLK * (NW - 1)


def _copy_block(prefix_hbm, ctx_hbm, suffix_hbm, out_hbm, sem, base, rows):
    cps = (
        pltpu.make_async_copy(
            prefix_hbm.at[pl.ds(base, rows)],
            out_hbm.at[pl.ds(base, rows), pl.ds(0, PRE_W)],
            sem,
        ),
        pltpu.make_async_copy(
            ctx_hbm.at[pl.ds(0, rows)],
            out_hbm.at[pl.ds(base, rows), pl.ds(PRE_W, CTX_W)],
            sem,
        ),
        pltpu.make_async_copy(
            suffix_hbm.at[pl.ds(base, rows)],
            out_hbm.at[pl.ds(base, rows), pl.ds(PRE_W + CTX_W, SUF_W)],
            sem,
        ),
    )
    for cp in cps:
        cp.start()
    for cp in cps:
        cp.wait()


def _sc_body(prefix_hbm, ctx_hbm, suffix_hbm, out_hbm, sem):
    c = lax.axis_index("c")
    s = lax.axis_index("s")
    wid = s * 2 + c

    @pl.when(wid < NW - 1)
    def _():
        _copy_block(prefix_hbm, ctx_hbm, suffix_hbm, out_hbm, sem, wid * BLK, BLK)

    @pl.when(wid == NW - 1)
    def _():
        _copy_block(
            prefix_hbm, ctx_hbm, suffix_hbm, out_hbm, sem, (NW - 1) * BLK, TAIL
        )


def kernel(token_prefix, ctx, token_suffix):
    prefix2d = token_prefix.reshape(N_CLS, PRE_W)
    suffix2d = token_suffix.reshape(N_CLS, SUF_W)
    ctx_rep = jnp.tile(ctx.reshape(1, CTX_W), (BLK, 1))
    out2d = pl.kernel(
        _sc_body,
        out_type=jax.ShapeDtypeStruct((N_CLS, ROW_W), jnp.float32),
        mesh=plsc.VectorSubcoreMesh(core_axis_name="c", subcore_axis_name="s"),
        scratch_types=[pltpu.SemaphoreType.DMA],
    )(prefix2d, ctx_rep, suffix2d)
    return out2d.reshape(N_CLS, 1 + N_CTX + SUF, DIM)
